# baseline (speedup 1.0000x reference)
import jax
import jax.numpy as jnp
from jax import lax
from jax.experimental import pallas as pl
from jax.experimental.pallas import tpu as pltpu
from jax.experimental.pallas import tpu_sc as plsc

N = 10000
C = 128
S = 4
K = 16
NPAD = 10240
BKN = 1024
BIG = 1e30

NW = 32
NODES_PER_W = NPAD // NW
GCH = 8
NCHUNK = NODES_PER_W // GCH


def _coords_body(xbf_ref, wst_ref, s_ref, sq_ref):
    s = jnp.dot(xbf_ref[...], wst_ref[...], preferred_element_type=jnp.float32)
    s_ref[...] = s
    sq_ref[...] = jnp.sum(s * s, axis=1, keepdims=True)


def _stage_coords(xbf, wstbf):
    return pl.pallas_call(
        _coords_body,
        out_shape=[
            jax.ShapeDtypeStruct((NPAD, 8), jnp.float32),
            jax.ShapeDtypeStruct((NPAD, 1), jnp.float32),
        ],
    )(xbf, wstbf)


def _knn_body(xbf_ref, wht_ref, bh_ref, sbf_ref, sbft_ref, sq_ref, sqrow_ref,
              h_ref, nbr_ref, d2_ref):
    i = pl.program_id(0)
    h_ref[...] = (jnp.dot(xbf_ref[...], wht_ref[...],
                          preferred_element_type=jnp.float32) + bh_ref[...])

    sbf_c = sbf_ref[pl.ds(i * BKN, BKN), :]
    mm = jnp.dot(sbf_c, sbft_ref[...], preferred_element_type=jnp.float32)
    a = sq_ref[pl.ds(i * BKN, BKN), :] + sqrow_ref[...]
    d2 = a - 2.0 * mm
    ci = lax.broadcasted_iota(jnp.int32, (BKN, NPAD), 1)
    d2_ref[...] = jnp.where(ci >= N, BIG, d2)

    ki = lax.broadcasted_iota(jnp.int32, (BKN, K), 1)

    def it(k, nbr):
        d2v = d2_ref[...]
        m = jnp.min(d2v, axis=1, keepdims=True)
        idx = jnp.min(jnp.where(d2v <= m, ci, NPAD), axis=1, keepdims=True)
        d2_ref[...] = jnp.where(ci == idx, BIG, d2v)
        return jnp.where(ki == k, jnp.broadcast_to(idx, (BKN, K)), nbr)

    nbr_ref[...] = lax.fori_loop(0, K, it, jnp.zeros((BKN, K), jnp.int32))


def _stage_knn(xbf, whtbf, bh2, sbf, sbft, sq_col, sqrow):
    grid = NPAD // BKN
    return pl.pallas_call(
        _knn_body,
        grid=(grid,),
        in_specs=[
            pl.BlockSpec((BKN, C), lambda i: (i, 0)),
            pl.BlockSpec((C, C), lambda i: (0, 0)),
            pl.BlockSpec((1, C), lambda i: (0, 0)),
            pl.BlockSpec((NPAD, 8), lambda i: (0, 0)),
            pl.BlockSpec((8, NPAD), lambda i: (0, 0)),
            pl.BlockSpec((NPAD, 1), lambda i: (0, 0)),
            pl.BlockSpec((1, NPAD), lambda i: (0, 0)),
        ],
        out_specs=[
            pl.BlockSpec((BKN, C), lambda i: (i, 0)),
            pl.BlockSpec((BKN, K), lambda i: (i, 0)),
        ],
        out_shape=[
            jax.ShapeDtypeStruct((NPAD, C), jnp.float32),
            jax.ShapeDtypeStruct((NPAD, K), jnp.int32),
        ],
        scratch_shapes=[pltpu.VMEM((BKN, NPAD), jnp.float32)],
    )(xbf, whtbf, bh2, sbf, sbft, sq_col, sqrow)


def _sc_body(nbr_hbm, s_hbm, h_hbm, mean_hbm, max_hbm,
             idx_v, s_v, rows_v, w_v, om_v, ox_v, sem):
    wid = lax.axis_index("s") * 2 + lax.axis_index("c")
    node0 = wid * NODES_PER_W
    pltpu.sync_copy(s_hbm, s_v)

    def chunk_body(cc, carry):
        nodebase = node0 + cc * GCH
        rowbase = nodebase * K
        pltpu.sync_copy(nbr_hbm.at[pl.ds(rowbase, GCH * K)], idx_v)
        cp = pltpu.async_copy(h_hbm.at[idx_v], rows_v, sem)
        cp.wait()

        def w_body(n, carry2):
            node = nodebase + n
            rbase = n * K
            idx16 = idx_v[pl.ds(rbase, K)]
            selfv = jnp.full((K,), 0, jnp.int32) + node
            acc = jnp.zeros((K,), jnp.float32)
            for c in range(S):
                ga = plsc.load_gather(s_v, [idx16 * 8 + c])
                gb = plsc.load_gather(s_v, [selfv * 8 + c])
                dcol = ga - gb
                acc = acc + dcol * dcol
            w_v[pl.ds(rbase, K)] = jnp.exp(-(acc + 1e-6))
            return carry2

        lax.fori_loop(0, GCH, w_body, 0)

        def node_body(n, carry2):
            rbase = n * K
            wsp = [plsc.load_gather(w_v, [jnp.full((K,), rbase + kk, jnp.int32)])
                   for kk in range(K)]
            for j in range(C // 16):
                t = wsp[0] * rows_v[rbase, pl.ds(j * 16, 16)]
                accm = t
                accx = t
                for kk in range(1, K):
                    t = wsp[kk] * rows_v[rbase + kk, pl.ds(j * 16, 16)]
                    accm = accm + t
                    accx = jnp.maximum(accx, t)
                om_v[pl.ds(n * C + j * 16, 16)] = accm * jnp.float32(1.0 / K)
                ox_v[pl.ds(n * C + j * 16, 16)] = accx
            return carry2

        lax.fori_loop(0, GCH, node_body, 0)
        pltpu.sync_copy(om_v, mean_hbm.at[pl.ds(nodebase * C, GCH * C)])
        pltpu.sync_copy(ox_v, max_hbm.at[pl.ds(nodebase * C, GCH * C)])
        return carry

    lax.fori_loop(0, NCHUNK, chunk_body, 0)


def _stage_gather(nbr_flat, s_flat, h_pad):
    mesh = plsc.VectorSubcoreMesh(core_axis_name="c", subcore_axis_name="s")
    f = pl.kernel(
        _sc_body,
        out_type=[
            jax.ShapeDtypeStruct((NPAD * C,), jnp.float32),
            jax.ShapeDtypeStruct((NPAD * C,), jnp.float32),
        ],
        mesh=mesh,
        compiler_params=pltpu.CompilerParams(needs_layout_passes=False),
        scratch_types=[
            pltpu.VMEM((GCH * K,), jnp.int32),
            pltpu.VMEM((NPAD * 8,), jnp.float32),
            pltpu.VMEM((GCH * K, C), jnp.float32),
            pltpu.VMEM((GCH * K,), jnp.float32),
            pltpu.VMEM((GCH * C,), jnp.float32),
            pltpu.VMEM((GCH * C,), jnp.float32),
            pltpu.SemaphoreType.DMA,
        ],
    )
    return f(nbr_flat, s_flat, h_pad)


def _tail_body(mean_ref, max_ref, x_ref, wlt_ref, blin_ref, wp1t_ref, bp1_ref,
               wp2t_ref, bp2_ref, g2_ref, b2_ref, g3_ref, b3_ref, out_ref):
    bf = jnp.bfloat16
    x = x_ref[...]
    xgn = (jnp.dot(mean_ref[...].astype(bf), wlt_ref[0:C, :],
                   preferred_element_type=jnp.float32)
           + jnp.dot(max_ref[...].astype(bf), wlt_ref[C:2 * C, :],
                     preferred_element_type=jnp.float32)
           + jnp.dot(x.astype(bf), wlt_ref[2 * C:3 * C, :],
                     preferred_element_type=jnp.float32)
           + blin_ref[...])
    x1 = xgn + x
    mu = jnp.mean(x1, axis=0, keepdims=True)
    var = jnp.mean((x1 - mu) ** 2, axis=0, keepdims=True)
    xh = (x1 - mu) * lax.rsqrt(var + 1e-5) * g2_ref[...] + b2_ref[...]
    z = (jnp.dot(xh.astype(bf), wp1t_ref[...],
                 preferred_element_type=jnp.float32) + bp1_ref[...])
    e = jnp.where(z > 0, z, jnp.exp(z) - 1.0)
    y = (jnp.dot(e.astype(bf), wp2t_ref[...],
                 preferred_element_type=jnp.float32) + bp2_ref[...])
    x2 = xh + y
    mu2 = jnp.mean(x2, axis=0, keepdims=True)
    var2 = jnp.mean((x2 - mu2) ** 2, axis=0, keepdims=True)
    out_ref[...] = ((x2 - mu2) * lax.rsqrt(var2 + 1e-5) * g3_ref[...]
                    + b3_ref[...])


def _stage_tail(mean_, max_, x, wltbf, blin2, wp1tbf, bp1_2, wp2tbf, bp2_2,
                g2, b2, g3, b3):
    return pl.pallas_call(
        _tail_body,
        out_shape=jax.ShapeDtypeStruct((N, C), jnp.float32),
    )(mean_, max_, x, wltbf, blin2, wp1tbf, bp1_2, wp2tbf, bp2_2, g2, b2, g3, b3)


def kernel(x, batch, original_coords, Ws, Wh, bh, Wlin, blin, Wp1, bp1, Wp2,
           bp2, gamma2, beta2, gamma3, beta3, step_count, num_layer):
    bf = jnp.bfloat16
    xbf = jnp.pad(x, ((0, NPAD - N), (0, 0))).astype(bf)
    wstbf = jnp.pad(Ws.T, ((0, 0), (0, 8 - S))).astype(bf)
    s, sq_col = _stage_coords(xbf, wstbf)

    sbf = s.astype(bf)
    sbft = sbf.T
    sqrow = sq_col.reshape(1, NPAD)
    h, nbr = _stage_knn(xbf, Wh.T.astype(bf), bh.reshape(1, C),
                        sbf, sbft, sq_col, sqrow)

    mean_flat, max_flat = _stage_gather(nbr.reshape(-1), s.reshape(-1), h)
    mean_ = mean_flat.reshape(NPAD, C)[:N]
    max_ = max_flat.reshape(NPAD, C)[:N]

    return _stage_tail(
        mean_, max_, x, Wlin.T.astype(bf), blin.reshape(1, C),
        Wp1.T.astype(bf), bp1.reshape(1, C), Wp2.T.astype(bf),
        bp2.reshape(1, C), gamma2.reshape(1, C), beta2.reshape(1, C),
        gamma3.reshape(1, C), beta3.reshape(1, C))


# speedup vs baseline: 1.0092x; 1.0092x over previous
import jax
import jax.numpy as jnp
from jax import lax
from jax.experimental import pallas as pl
from jax.experimental.pallas import tpu as pltpu
from jax.experimental.pallas import tpu_sc as plsc

N = 10000
C = 128
S = 4
K = 16
NPAD = 10240
BKN = 512
BIG = 1e30

NW = 32
NODES_PER_W = NPAD // NW
GCH = 8
NCHUNK = NODES_PER_W // GCH


def _coords_body(xbf_ref, wst_ref, s_ref, sq_ref):
    s = jnp.dot(xbf_ref[...], wst_ref[...], preferred_element_type=jnp.float32)
    s_ref[...] = s
    sq_ref[...] = jnp.sum(s * s, axis=1, keepdims=True)


def _stage_coords(xbf, wstbf):
    return pl.pallas_call(
        _coords_body,
        out_shape=[
            jax.ShapeDtypeStruct((NPAD, 8), jnp.float32),
            jax.ShapeDtypeStruct((NPAD, 1), jnp.float32),
        ],
    )(xbf, wstbf)


def _knn_body(xbf_ref, wht_ref, bh_ref, sbf_ref, sbft_ref, sq_ref, sqrow_ref,
              h_ref, nbr_ref, d2_ref):
    i = pl.program_id(0)
    h_ref[...] = (jnp.dot(xbf_ref[...], wht_ref[...],
                          preferred_element_type=jnp.float32) + bh_ref[...])

    sbf_c = sbf_ref[pl.ds(i * BKN, BKN), :]
    mm = jnp.dot(sbf_c, sbft_ref[...], preferred_element_type=jnp.float32)
    a = sq_ref[pl.ds(i * BKN, BKN), :] + sqrow_ref[...]
    d2 = a - 2.0 * mm
    ci = lax.broadcasted_iota(jnp.int32, (BKN, NPAD), 1)
    d2_ref[...] = jnp.where(ci >= N, BIG, d2)

    ki = lax.broadcasted_iota(jnp.int32, (BKN, K), 1)

    def it(k, nbr):
        d2v = d2_ref[...]
        m = jnp.min(d2v, axis=1, keepdims=True)
        idx = jnp.min(jnp.where(d2v <= m, ci, NPAD), axis=1, keepdims=True)
        d2_ref[...] = jnp.where(ci == idx, BIG, d2v)
        return jnp.where(ki == k, jnp.broadcast_to(idx, (BKN, K)), nbr)

    nbr_ref[...] = lax.fori_loop(0, K, it, jnp.zeros((BKN, K), jnp.int32))


def _stage_knn(xbf, whtbf, bh2, sbf, sbft, sq_col, sqrow):
    grid = NPAD // BKN
    return pl.pallas_call(
        _knn_body,
        grid=(grid,),
        in_specs=[
            pl.BlockSpec((BKN, C), lambda i: (i, 0)),
            pl.BlockSpec((C, C), lambda i: (0, 0)),
            pl.BlockSpec((1, C), lambda i: (0, 0)),
            pl.BlockSpec((NPAD, 8), lambda i: (0, 0)),
            pl.BlockSpec((8, NPAD), lambda i: (0, 0)),
            pl.BlockSpec((NPAD, 1), lambda i: (0, 0)),
            pl.BlockSpec((1, NPAD), lambda i: (0, 0)),
        ],
        out_specs=[
            pl.BlockSpec((BKN, C), lambda i: (i, 0)),
            pl.BlockSpec((BKN, K), lambda i: (i, 0)),
        ],
        out_shape=[
            jax.ShapeDtypeStruct((NPAD, C), jnp.float32),
            jax.ShapeDtypeStruct((NPAD, K), jnp.int32),
        ],
        scratch_shapes=[pltpu.VMEM((BKN, NPAD), jnp.float32)],
    )(xbf, whtbf, bh2, sbf, sbft, sq_col, sqrow)


def _sc_body(nbr_hbm, s_hbm, h_hbm, mean_hbm, max_hbm,
             idx_v, s_v, rows_v, w_v, om_v, ox_v, sem):
    wid = lax.axis_index("s") * 2 + lax.axis_index("c")
    node0 = wid * NODES_PER_W
    pltpu.sync_copy(s_hbm, s_v)

    def chunk_body(cc, carry):
        nodebase = node0 + cc * GCH
        rowbase = nodebase * K
        pltpu.sync_copy(nbr_hbm.at[pl.ds(rowbase, GCH * K)], idx_v)
        cp = pltpu.async_copy(h_hbm.at[idx_v], rows_v, sem)
        cp.wait()

        def w_body(n, carry2):
            node = nodebase + n
            rbase = n * K
            idx16 = idx_v[pl.ds(rbase, K)]
            selfv = jnp.full((K,), 0, jnp.int32) + node
            acc = jnp.zeros((K,), jnp.float32)
            for c in range(S):
                ga = plsc.load_gather(s_v, [idx16 * 8 + c])
                gb = plsc.load_gather(s_v, [selfv * 8 + c])
                dcol = ga - gb
                acc = acc + dcol * dcol
            w_v[pl.ds(rbase, K)] = jnp.exp(-(acc + 1e-6))
            return carry2

        lax.fori_loop(0, GCH, w_body, 0)

        def node_body(n, carry2):
            rbase = n * K
            wsp = [plsc.load_gather(w_v, [jnp.full((K,), rbase + kk, jnp.int32)])
                   for kk in range(K)]
            for j in range(C // 16):
                t = wsp[0] * rows_v[rbase, pl.ds(j * 16, 16)]
                accm = t
                accx = t
                for kk in range(1, K):
                    t = wsp[kk] * rows_v[rbase + kk, pl.ds(j * 16, 16)]
                    accm = accm + t
                    accx = jnp.maximum(accx, t)
                om_v[pl.ds(n * C + j * 16, 16)] = accm * jnp.float32(1.0 / K)
                ox_v[pl.ds(n * C + j * 16, 16)] = accx
            return carry2

        lax.fori_loop(0, GCH, node_body, 0)
        pltpu.sync_copy(om_v, mean_hbm.at[pl.ds(nodebase * C, GCH * C)])
        pltpu.sync_copy(ox_v, max_hbm.at[pl.ds(nodebase * C, GCH * C)])
        return carry

    lax.fori_loop(0, NCHUNK, chunk_body, 0)


def _stage_gather(nbr_flat, s_flat, h_pad):
    mesh = plsc.VectorSubcoreMesh(core_axis_name="c", subcore_axis_name="s")
    f = pl.kernel(
        _sc_body,
        out_type=[
            jax.ShapeDtypeStruct((NPAD * C,), jnp.float32),
            jax.ShapeDtypeStruct((NPAD * C,), jnp.float32),
        ],
        mesh=mesh,
        compiler_params=pltpu.CompilerParams(needs_layout_passes=False),
        scratch_types=[
            pltpu.VMEM((GCH * K,), jnp.int32),
            pltpu.VMEM((NPAD * 8,), jnp.float32),
            pltpu.VMEM((GCH * K, C), jnp.float32),
            pltpu.VMEM((GCH * K,), jnp.float32),
            pltpu.VMEM((GCH * C,), jnp.float32),
            pltpu.VMEM((GCH * C,), jnp.float32),
            pltpu.SemaphoreType.DMA,
        ],
    )
    return f(nbr_flat, s_flat, h_pad)


def _tail_body(mean_ref, max_ref, x_ref, wlt_ref, blin_ref, wp1t_ref, bp1_ref,
               wp2t_ref, bp2_ref, g2_ref, b2_ref, g3_ref, b3_ref, out_ref):
    bf = jnp.bfloat16
    x = x_ref[...]
    xgn = (jnp.dot(mean_ref[...].astype(bf), wlt_ref[0:C, :],
                   preferred_element_type=jnp.float32)
           + jnp.dot(max_ref[...].astype(bf), wlt_ref[C:2 * C, :],
                     preferred_element_type=jnp.float32)
           + jnp.dot(x.astype(bf), wlt_ref[2 * C:3 * C, :],
                     preferred_element_type=jnp.float32)
           + blin_ref[...])
    x1 = xgn + x
    mu = jnp.mean(x1, axis=0, keepdims=True)
    var = jnp.mean((x1 - mu) ** 2, axis=0, keepdims=True)
    xh = (x1 - mu) * lax.rsqrt(var + 1e-5) * g2_ref[...] + b2_ref[...]
    z = (jnp.dot(xh.astype(bf), wp1t_ref[...],
                 preferred_element_type=jnp.float32) + bp1_ref[...])
    e = jnp.where(z > 0, z, jnp.exp(z) - 1.0)
    y = (jnp.dot(e.astype(bf), wp2t_ref[...],
                 preferred_element_type=jnp.float32) + bp2_ref[...])
    x2 = xh + y
    mu2 = jnp.mean(x2, axis=0, keepdims=True)
    var2 = jnp.mean((x2 - mu2) ** 2, axis=0, keepdims=True)
    out_ref[...] = ((x2 - mu2) * lax.rsqrt(var2 + 1e-5) * g3_ref[...]
                    + b3_ref[...])


def _stage_tail(mean_, max_, x, wltbf, blin2, wp1tbf, bp1_2, wp2tbf, bp2_2,
                g2, b2, g3, b3):
    return pl.pallas_call(
        _tail_body,
        out_shape=jax.ShapeDtypeStruct((N, C), jnp.float32),
    )(mean_, max_, x, wltbf, blin2, wp1tbf, bp1_2, wp2tbf, bp2_2, g2, b2, g3, b3)


def kernel(x, batch, original_coords, Ws, Wh, bh, Wlin, blin, Wp1, bp1, Wp2,
           bp2, gamma2, beta2, gamma3, beta3, step_count, num_layer):
    bf = jnp.bfloat16
    xbf = jnp.pad(x, ((0, NPAD - N), (0, 0))).astype(bf)
    wstbf = jnp.pad(Ws.T, ((0, 0), (0, 8 - S))).astype(bf)
    s, sq_col = _stage_coords(xbf, wstbf)

    sbf = s.astype(bf)
    sbft = sbf.T
    sqrow = sq_col.reshape(1, NPAD)
    h, nbr = _stage_knn(xbf, Wh.T.astype(bf), bh.reshape(1, C),
                        sbf, sbft, sq_col, sqrow)

    mean_flat, max_flat = _stage_gather(nbr.reshape(-1), s.reshape(-1), h)
    mean_ = mean_flat.reshape(NPAD, C)[:N]
    max_ = max_flat.reshape(NPAD, C)[:N]

    return _stage_tail(
        mean_, max_, x, Wlin.T.astype(bf), blin.reshape(1, C),
        Wp1.T.astype(bf), bp1.reshape(1, C), Wp2.T.astype(bf),
        bp2.reshape(1, C), gamma2.reshape(1, C), beta2.reshape(1, C),
        gamma3.reshape(1, C), beta3.reshape(1, C))
